# Initial kernel scaffold; baseline (speedup 1.0000x reference)
import jax
import jax.numpy as jnp
from jax.experimental import pallas as pl


def kernel(x, edge_index, Wl0, Wr0, Wl1, Wr1, Wl2, Wr2, bl2, g0, bn0, g1, bn1, Wc, bc):
    raise NotImplementedError("write your pallas kernel here")


# speedup vs baseline: 2.8070x; 2.8070x over previous
import functools

import jax
import jax.numpy as jnp
from jax import lax
from jax.experimental import pallas as pl
from jax.experimental.pallas import tpu as pltpu
from jax.experimental.pallas import tpu_sc as plsc

N = 10000
E = 320000
D = 128
OUT = 40

NC = 2
NS = 16
NW = NC * NS
CHUNK = 128
CPW = 80
E_PAD = NW * CPW * CHUNK
N_T = 10240
ROWS_PER_TILE = N_T // NS
INV_BN = 1.0 / (1.0 + 1e-5) ** 0.5


def _sc_segsum_body(p_hbm, srcg, dstg, z128, out_hbm,
                    src_v, dst_v, rows_v, sem, acc):
    c = lax.axis_index("c")
    s = lax.axis_index("s")
    wid = c * NS + s
    pltpu.sync_copy(srcg.at[wid], src_v)
    pltpu.sync_copy(dstg.at[wid], dst_v)
    r0 = s * ROWS_PER_TILE
    pltpu.sync_copy(z128.at[pl.ds(r0, ROWS_PER_TILE)],
                    acc.at[pl.ds(r0, ROWS_PER_TILE)])
    plsc.subcore_barrier()

    def body(j, carry):
        pltpu.async_copy(p_hbm.at[src_v.at[j]], rows_v, sem).wait()
        pltpu.sync_copy(rows_v, acc.at[dst_v.at[j]], add=True)
        return carry

    lax.fori_loop(0, CPW, body, 0)
    plsc.subcore_barrier()
    pltpu.sync_copy(acc.at[pl.ds(r0, ROWS_PER_TILE)],
                    out_hbm.at[c, pl.ds(r0, ROWS_PER_TILE)])


def _sc_degree_body(dstg, z128, ones128, deg_hbm, dst_v, ones_v, dega):
    c = lax.axis_index("c")
    s = lax.axis_index("s")
    wid = c * NS + s
    pltpu.sync_copy(dstg.at[wid], dst_v)
    pltpu.sync_copy(ones128, ones_v)
    r0 = s * ROWS_PER_TILE
    pltpu.sync_copy(z128.at[pl.ds(r0, ROWS_PER_TILE)],
                    dega.at[pl.ds(r0, ROWS_PER_TILE)])
    plsc.subcore_barrier()

    def body(j, carry):
        pltpu.sync_copy(ones_v, dega.at[dst_v.at[j]], add=True)
        return carry

    lax.fori_loop(0, CPW, body, 0)
    plsc.subcore_barrier()
    pltpu.sync_copy(dega.at[pl.ds(r0, ROWS_PER_TILE)],
                    deg_hbm.at[c, pl.ds(r0, ROWS_PER_TILE)])


@functools.cache
def _sc_kernels():
    mesh = plsc.VectorSubcoreMesh(
        core_axis_name="c", subcore_axis_name="s",
        num_cores=NC, num_subcores=NS)
    segsum = pl.kernel(
        _sc_segsum_body,
        out_type=jax.ShapeDtypeStruct((NC, N_T, D), jnp.float32),
        mesh=mesh,
        scratch_types=[
            pltpu.VMEM((CPW, CHUNK), jnp.int32),
            pltpu.VMEM((CPW, CHUNK), jnp.int32),
            pltpu.VMEM((CHUNK, D), jnp.float32),
            pltpu.SemaphoreType.DMA,
            pltpu.VMEM_SHARED((N_T, D), jnp.float32),
        ],
    )
    degree = pl.kernel(
        _sc_degree_body,
        out_type=jax.ShapeDtypeStruct((NC, N_T, D), jnp.float32),
        mesh=mesh,
        scratch_types=[
            pltpu.VMEM((CPW, CHUNK), jnp.int32),
            pltpu.VMEM((CHUNK, D), jnp.float32),
            pltpu.VMEM_SHARED((N_T, D), jnp.float32),
        ],
    )
    return segsum, degree


def _sc_segsum(p, srcg, dstg, z128):
    return _sc_kernels()[0](p, srcg, dstg, z128)


def _sc_degree(dstg, z128, ones128):
    return _sc_kernels()[1](dstg, z128, ones128)


_R = 1024
_GRID = N_T // _R


def _tc_project_body(x_ref, wl_ref, wr_ref, p_ref, q_ref):
    xv = x_ref[...]
    p_ref[...] = jnp.dot(xv, wl_ref[...], preferred_element_type=jnp.float32)
    q_ref[...] = jnp.dot(xv, wr_ref[...], preferred_element_type=jnp.float32)


def _tc_project(x, wl, wr):
    return pl.pallas_call(
        _tc_project_body,
        grid=(_GRID,),
        in_specs=[
            pl.BlockSpec((_R, D), lambda i: (i, 0)),
            pl.BlockSpec((D, D), lambda i: (0, 0)),
            pl.BlockSpec((D, D), lambda i: (0, 0)),
        ],
        out_specs=[
            pl.BlockSpec((_R, D), lambda i: (i, 0)),
            pl.BlockSpec((_R, D), lambda i: (i, 0)),
        ],
        out_shape=[jax.ShapeDtypeStruct((N_T, D), jnp.float32),
                   jax.ShapeDtypeStruct((N_T, D), jnp.float32)],
    )(x, wl, wr)


def _tc_finish_body(a0_ref, a1_ref, d0_ref, d1_ref, q_ref, g_ref, b_ref,
                    wl_ref, wr_ref, h_ref, p_ref, qn_ref):
    dsum = d0_ref[...] + d1_ref[...]
    invd = 1.0 / jnp.maximum(dsum[:, 0:1], 1.0)
    mean = (a0_ref[...] + a1_ref[...]) * invd
    u = (mean + q_ref[...]) * INV_BN * g_ref[...] + b_ref[...]
    hv = jnp.maximum(u, 0.0)
    h_ref[...] = hv
    p_ref[...] = jnp.dot(hv, wl_ref[...], preferred_element_type=jnp.float32)
    qn_ref[...] = jnp.dot(hv, wr_ref[...], preferred_element_type=jnp.float32)


def _tc_finish(a0, a1, d0, d1, q, g, b, wl, wr):
    return pl.pallas_call(
        _tc_finish_body,
        grid=(_GRID,),
        in_specs=[
            pl.BlockSpec((_R, D), lambda i: (i, 0)),
            pl.BlockSpec((_R, D), lambda i: (i, 0)),
            pl.BlockSpec((_R, D), lambda i: (i, 0)),
            pl.BlockSpec((_R, D), lambda i: (i, 0)),
            pl.BlockSpec((_R, D), lambda i: (i, 0)),
            pl.BlockSpec((1, D), lambda i: (0, 0)),
            pl.BlockSpec((1, D), lambda i: (0, 0)),
            pl.BlockSpec((D, D), lambda i: (0, 0)),
            pl.BlockSpec((D, D), lambda i: (0, 0)),
        ],
        out_specs=[
            pl.BlockSpec((_R, D), lambda i: (i, 0)),
            pl.BlockSpec((_R, D), lambda i: (i, 0)),
            pl.BlockSpec((_R, D), lambda i: (i, 0)),
        ],
        out_shape=[jax.ShapeDtypeStruct((N_T, D), jnp.float32),
                   jax.ShapeDtypeStruct((N_T, D), jnp.float32),
                   jax.ShapeDtypeStruct((N_T, D), jnp.float32)],
    )(a0, a1, d0, d1, q, g, b, wl, wr)


def _tc_final_body(a0_ref, a1_ref, d0_ref, d1_ref, q_ref, bl_ref,
                   h1_ref, h2_ref, wc1_ref, wc2_ref, wc3_ref, bc_ref, o_ref):
    dsum = d0_ref[...] + d1_ref[...]
    invd = 1.0 / jnp.maximum(dsum[:, 0:1], 1.0)
    h3 = (a0_ref[...] + a1_ref[...]) * invd + q_ref[...] + bl_ref[...]
    o = jnp.dot(h1_ref[...], wc1_ref[...], preferred_element_type=jnp.float32)
    o += jnp.dot(h2_ref[...], wc2_ref[...], preferred_element_type=jnp.float32)
    o += jnp.dot(h3, wc3_ref[...], preferred_element_type=jnp.float32)
    o_ref[...] = o + bc_ref[...]


def _tc_final(a0, a1, d0, d1, q, bl, h1, h2, wc1, wc2, wc3, bc):
    return pl.pallas_call(
        _tc_final_body,
        grid=(_GRID,),
        in_specs=[
            pl.BlockSpec((_R, D), lambda i: (i, 0)),
            pl.BlockSpec((_R, D), lambda i: (i, 0)),
            pl.BlockSpec((_R, D), lambda i: (i, 0)),
            pl.BlockSpec((_R, D), lambda i: (i, 0)),
            pl.BlockSpec((_R, D), lambda i: (i, 0)),
            pl.BlockSpec((1, D), lambda i: (0, 0)),
            pl.BlockSpec((_R, D), lambda i: (i, 0)),
            pl.BlockSpec((_R, D), lambda i: (i, 0)),
            pl.BlockSpec((D, OUT), lambda i: (0, 0)),
            pl.BlockSpec((D, OUT), lambda i: (0, 0)),
            pl.BlockSpec((D, OUT), lambda i: (0, 0)),
            pl.BlockSpec((1, OUT), lambda i: (0, 0)),
        ],
        out_specs=pl.BlockSpec((_R, OUT), lambda i: (i, 0)),
        out_shape=jax.ShapeDtypeStruct((N_T, OUT), jnp.float32),
    )(a0, a1, d0, d1, q, bl, h1, h2, wc1, wc2, wc3, bc)


def kernel(x, edge_index, Wl0, Wr0, Wl1, Wr1, Wl2, Wr2, bl2, g0, bn0, g1, bn1,
           Wc, bc):
    src = edge_index[0]
    dst = edge_index[1]
    srcg = jnp.concatenate(
        [src, jnp.zeros((E_PAD - E,), jnp.int32)]).reshape(NW, CPW, CHUNK)
    dstg = jnp.concatenate(
        [dst, jnp.full((E_PAD - E,), N, jnp.int32)]).reshape(NW, CPW, CHUNK)
    z128 = jnp.zeros((N_T, D), jnp.float32)
    xp = jnp.concatenate([x, jnp.zeros((N_T - N, D), jnp.float32)])
    ones128 = jnp.ones((CHUNK, D), jnp.float32)

    g0r = g0.reshape(1, D)
    bn0r = bn0.reshape(1, D)
    g1r = g1.reshape(1, D)
    bn1r = bn1.reshape(1, D)
    bl2r = bl2.reshape(1, D)
    wc1, wc2, wc3 = Wc[:D], Wc[D:2 * D], Wc[2 * D:]
    bcr = bc.reshape(1, OUT)

    p0, q0 = _tc_project(xp, Wl0, Wr0)
    dd = _sc_degree(dstg, z128, ones128)
    a = _sc_segsum(p0, srcg, dstg, z128)
    d0p, d1p = dd[0], dd[1]
    h1, p1, q1 = _tc_finish(a[0], a[1], d0p, d1p, q0, g0r, bn0r, Wl1, Wr1)
    a = _sc_segsum(p1, srcg, dstg, z128)
    h2, p2, q2 = _tc_finish(a[0], a[1], d0p, d1p, q1, g1r, bn1r, Wl2, Wr2)
    a = _sc_segsum(p2, srcg, dstg, z128)
    out = _tc_final(a[0], a[1], d0p, d1p, q2, bl2r, h1, h2, wc1, wc2, wc3,
                    bcr)
    return out[:N]


# speedup vs baseline: 3.1138x; 1.1093x over previous
import functools

import jax
import jax.numpy as jnp
from jax import lax
from jax.experimental import pallas as pl
from jax.experimental.pallas import tpu as pltpu
from jax.experimental.pallas import tpu_sc as plsc

N = 10000
E = 320000
D = 128
OUT = 40

NC = 2
NS = 16
NW = NC * NS
CHUNK = 128
CPW = 80
HPW = CPW // 2
E_PAD = NW * CPW * CHUNK
N_T = 10240
ROWS_PER_TILE = N_T // NS
INV_BN = 1.0 / (1.0 + 1e-5) ** 0.5


NBUF = 2


def _sc_segsum_body(p_hbm, srcg, dstg, z128, out_hbm,
                    src_v, dst_v, rows0, rows1,
                    sem0, sem1, acc):
    c = lax.axis_index("c")
    s = lax.axis_index("s")
    wid = c * NS + s
    r0 = s * ROWS_PER_TILE
    pltpu.sync_copy(z128.at[pl.ds(r0, ROWS_PER_TILE)],
                    acc.at[pl.ds(r0, ROWS_PER_TILE)])
    plsc.subcore_barrier()

    bufs = (rows0, rows1)
    sems = (sem0, sem1)
    for h in range(2):
        pltpu.sync_copy(srcg.at[wid, pl.ds(h * HPW, HPW)], src_v)
        pltpu.sync_copy(dstg.at[wid, pl.ds(h * HPW, HPW)], dst_v)
        for b in range(NBUF):
            pltpu.make_async_copy(
                p_hbm.at[src_v.at[b]], bufs[b], sems[b]).start()

        def body(g, carry):
            j = g * NBUF
            for b in range(NBUF):
                pltpu.make_async_copy(
                    p_hbm.at[src_v.at[j + b]], bufs[b], sems[b]).wait()
                pltpu.sync_copy(bufs[b], acc.at[dst_v.at[j + b]], add=True)
                pltpu.make_async_copy(
                    p_hbm.at[src_v.at[j + b + NBUF]], bufs[b], sems[b]).start()
            return carry

        lax.fori_loop(0, HPW // NBUF - 1, body, 0)
        jt = HPW - NBUF
        for b in range(NBUF):
            pltpu.make_async_copy(
                p_hbm.at[src_v.at[jt + b]], bufs[b], sems[b]).wait()
            pltpu.sync_copy(bufs[b], acc.at[dst_v.at[jt + b]], add=True)
    plsc.subcore_barrier()
    pltpu.sync_copy(acc.at[pl.ds(r0, ROWS_PER_TILE)],
                    out_hbm.at[c, pl.ds(r0, ROWS_PER_TILE)])


def _sc_degree_body(dstg, z128, ones128, deg_hbm, dst_v, ones_v, dega):
    c = lax.axis_index("c")
    s = lax.axis_index("s")
    wid = c * NS + s
    pltpu.sync_copy(dstg.at[wid], dst_v)
    pltpu.sync_copy(ones128, ones_v)
    r0 = s * ROWS_PER_TILE
    pltpu.sync_copy(z128.at[pl.ds(r0, ROWS_PER_TILE)],
                    dega.at[pl.ds(r0, ROWS_PER_TILE)])
    plsc.subcore_barrier()

    def body(j, carry):
        pltpu.sync_copy(ones_v, dega.at[dst_v.at[j]], add=True)
        return carry

    lax.fori_loop(0, CPW, body, 0)
    plsc.subcore_barrier()
    pltpu.sync_copy(dega.at[pl.ds(r0, ROWS_PER_TILE)],
                    deg_hbm.at[c, pl.ds(r0, ROWS_PER_TILE)])


@functools.cache
def _sc_kernels():
    mesh = plsc.VectorSubcoreMesh(
        core_axis_name="c", subcore_axis_name="s",
        num_cores=NC, num_subcores=NS)
    segsum = pl.kernel(
        _sc_segsum_body,
        out_type=jax.ShapeDtypeStruct((NC, N_T, D), jnp.float32),
        mesh=mesh,
        scratch_types=[
            pltpu.VMEM((HPW, CHUNK), jnp.int32),
            pltpu.VMEM((HPW, CHUNK), jnp.int32),
            pltpu.VMEM((CHUNK, D), jnp.float32),
            pltpu.VMEM((CHUNK, D), jnp.float32),
            pltpu.SemaphoreType.DMA,
            pltpu.SemaphoreType.DMA,
            pltpu.VMEM_SHARED((N_T, D), jnp.float32),
        ],
    )
    degree = pl.kernel(
        _sc_degree_body,
        out_type=jax.ShapeDtypeStruct((NC, N_T, D), jnp.float32),
        mesh=mesh,
        scratch_types=[
            pltpu.VMEM((CPW, CHUNK), jnp.int32),
            pltpu.VMEM((CHUNK, D), jnp.float32),
            pltpu.VMEM_SHARED((N_T, D), jnp.float32),
        ],
    )
    return segsum, degree


def _sc_segsum(p, srcg, dstg, z128):
    return _sc_kernels()[0](p, srcg, dstg, z128)


def _sc_degree(dstg, z128, ones128):
    return _sc_kernels()[1](dstg, z128, ones128)


_R = 1024
_GRID = N_T // _R


def _tc_project_body(x_ref, wl_ref, wr_ref, p_ref, q_ref):
    xv = x_ref[...]
    p_ref[...] = jnp.dot(xv, wl_ref[...], preferred_element_type=jnp.float32)
    q_ref[...] = jnp.dot(xv, wr_ref[...], preferred_element_type=jnp.float32)


def _tc_project(x, wl, wr):
    return pl.pallas_call(
        _tc_project_body,
        grid=(_GRID,),
        in_specs=[
            pl.BlockSpec((_R, D), lambda i: (i, 0)),
            pl.BlockSpec((D, D), lambda i: (0, 0)),
            pl.BlockSpec((D, D), lambda i: (0, 0)),
        ],
        out_specs=[
            pl.BlockSpec((_R, D), lambda i: (i, 0)),
            pl.BlockSpec((_R, D), lambda i: (i, 0)),
        ],
        out_shape=[jax.ShapeDtypeStruct((N_T, D), jnp.float32),
                   jax.ShapeDtypeStruct((N_T, D), jnp.float32)],
    )(x, wl, wr)


def _tc_finish_body(a0_ref, a1_ref, d0_ref, d1_ref, q_ref, g_ref, b_ref,
                    wl_ref, wr_ref, h_ref, p_ref, qn_ref):
    dsum = d0_ref[...] + d1_ref[...]
    invd = 1.0 / jnp.maximum(dsum[:, 0:1], 1.0)
    mean = (a0_ref[...] + a1_ref[...]) * invd
    u = (mean + q_ref[...]) * INV_BN * g_ref[...] + b_ref[...]
    hv = jnp.maximum(u, 0.0)
    h_ref[...] = hv
    p_ref[...] = jnp.dot(hv, wl_ref[...], preferred_element_type=jnp.float32)
    qn_ref[...] = jnp.dot(hv, wr_ref[...], preferred_element_type=jnp.float32)


def _tc_finish(a0, a1, d0, d1, q, g, b, wl, wr):
    return pl.pallas_call(
        _tc_finish_body,
        grid=(_GRID,),
        in_specs=[
            pl.BlockSpec((_R, D), lambda i: (i, 0)),
            pl.BlockSpec((_R, D), lambda i: (i, 0)),
            pl.BlockSpec((_R, D), lambda i: (i, 0)),
            pl.BlockSpec((_R, D), lambda i: (i, 0)),
            pl.BlockSpec((_R, D), lambda i: (i, 0)),
            pl.BlockSpec((1, D), lambda i: (0, 0)),
            pl.BlockSpec((1, D), lambda i: (0, 0)),
            pl.BlockSpec((D, D), lambda i: (0, 0)),
            pl.BlockSpec((D, D), lambda i: (0, 0)),
        ],
        out_specs=[
            pl.BlockSpec((_R, D), lambda i: (i, 0)),
            pl.BlockSpec((_R, D), lambda i: (i, 0)),
            pl.BlockSpec((_R, D), lambda i: (i, 0)),
        ],
        out_shape=[jax.ShapeDtypeStruct((N_T, D), jnp.float32),
                   jax.ShapeDtypeStruct((N_T, D), jnp.float32),
                   jax.ShapeDtypeStruct((N_T, D), jnp.float32)],
    )(a0, a1, d0, d1, q, g, b, wl, wr)


def _tc_final_body(a0_ref, a1_ref, d0_ref, d1_ref, q_ref, bl_ref,
                   h1_ref, h2_ref, wc1_ref, wc2_ref, wc3_ref, bc_ref, o_ref):
    dsum = d0_ref[...] + d1_ref[...]
    invd = 1.0 / jnp.maximum(dsum[:, 0:1], 1.0)
    h3 = (a0_ref[...] + a1_ref[...]) * invd + q_ref[...] + bl_ref[...]
    o = jnp.dot(h1_ref[...], wc1_ref[...], preferred_element_type=jnp.float32)
    o += jnp.dot(h2_ref[...], wc2_ref[...], preferred_element_type=jnp.float32)
    o += jnp.dot(h3, wc3_ref[...], preferred_element_type=jnp.float32)
    o_ref[...] = o + bc_ref[...]


def _tc_final(a0, a1, d0, d1, q, bl, h1, h2, wc1, wc2, wc3, bc):
    return pl.pallas_call(
        _tc_final_body,
        grid=(_GRID,),
        in_specs=[
            pl.BlockSpec((_R, D), lambda i: (i, 0)),
            pl.BlockSpec((_R, D), lambda i: (i, 0)),
            pl.BlockSpec((_R, D), lambda i: (i, 0)),
            pl.BlockSpec((_R, D), lambda i: (i, 0)),
            pl.BlockSpec((_R, D), lambda i: (i, 0)),
            pl.BlockSpec((1, D), lambda i: (0, 0)),
            pl.BlockSpec((_R, D), lambda i: (i, 0)),
            pl.BlockSpec((_R, D), lambda i: (i, 0)),
            pl.BlockSpec((D, OUT), lambda i: (0, 0)),
            pl.BlockSpec((D, OUT), lambda i: (0, 0)),
            pl.BlockSpec((D, OUT), lambda i: (0, 0)),
            pl.BlockSpec((1, OUT), lambda i: (0, 0)),
        ],
        out_specs=pl.BlockSpec((_R, OUT), lambda i: (i, 0)),
        out_shape=jax.ShapeDtypeStruct((N_T, OUT), jnp.float32),
    )(a0, a1, d0, d1, q, bl, h1, h2, wc1, wc2, wc3, bc)


def kernel(x, edge_index, Wl0, Wr0, Wl1, Wr1, Wl2, Wr2, bl2, g0, bn0, g1, bn1,
           Wc, bc):
    src = edge_index[0]
    dst = edge_index[1]
    srcg = jnp.concatenate(
        [src, jnp.zeros((E_PAD - E,), jnp.int32)]).reshape(NW, CPW, CHUNK)
    dstg = jnp.concatenate(
        [dst, jnp.full((E_PAD - E,), N, jnp.int32)]).reshape(NW, CPW, CHUNK)
    z128 = jnp.zeros((N_T, D), jnp.float32)
    xp = jnp.concatenate([x, jnp.zeros((N_T - N, D), jnp.float32)])
    ones128 = jnp.ones((CHUNK, D), jnp.float32)

    g0r = g0.reshape(1, D)
    bn0r = bn0.reshape(1, D)
    g1r = g1.reshape(1, D)
    bn1r = bn1.reshape(1, D)
    bl2r = bl2.reshape(1, D)
    wc1, wc2, wc3 = Wc[:D], Wc[D:2 * D], Wc[2 * D:]
    bcr = bc.reshape(1, OUT)

    p0, q0 = _tc_project(xp, Wl0, Wr0)
    dd = _sc_degree(dstg, z128, ones128)
    a = _sc_segsum(p0, srcg, dstg, z128)
    d0p, d1p = dd[0], dd[1]
    h1, p1, q1 = _tc_finish(a[0], a[1], d0p, d1p, q0, g0r, bn0r, Wl1, Wr1)
    a = _sc_segsum(p1, srcg, dstg, z128)
    h2, p2, q2 = _tc_finish(a[0], a[1], d0p, d1p, q1, g1r, bn1r, Wl2, Wr2)
    a = _sc_segsum(p2, srcg, dstg, z128)
    out = _tc_final(a[0], a[1], d0p, d1p, q2, bl2r, h1, h2, wc1, wc2, wc3,
                    bcr)
    return out[:N]
